# baseline (speedup 1.0000x reference)
import functools

import jax
import jax.numpy as jnp
from jax import lax
from jax.experimental import pallas as pl
from jax.experimental.pallas import tpu as pltpu
from jax.experimental.pallas import tpu_sc as plsc

BITS = 10
PATCH = 32
N = 8192
NB = 256
BINS = 256
NV = N // 16


def _codes_body(x_ref, o_ref):
    X0f, X1f, X2f = x_ref[0], x_ref[1], x_ref[2]
    maxv = (1 << BITS) - 1
    Xs = []
    for xf in (X0f, X1f, X2f):
        mn = xf.min(axis=-1, keepdims=True)
        mx = xf.max(axis=-1, keepdims=True)
        p = (xf - mn) / jnp.maximum(mx - mn, 1e-9)
        Xs.append(
            jnp.clip((p * (1 << BITS)).astype(jnp.int32), 0, maxv).astype(jnp.uint32)
        )
    X0, X1, X2 = Xs
    Q = 1 << (BITS - 1)
    while Q > 1:
        P = jnp.uint32(Q - 1)
        c0 = (X0 & Q) > 0
        X0 = jnp.where(c0, X0 ^ P, X0)
        c1 = (X1 & Q) > 0
        t = (X0 ^ X1) & P
        X0, X1 = jnp.where(c1, X0 ^ P, X0 ^ t), jnp.where(c1, X1, X1 ^ t)
        c2 = (X2 & Q) > 0
        t = (X0 ^ X2) & P
        X0, X2 = jnp.where(c2, X0 ^ P, X0 ^ t), jnp.where(c2, X2, X2 ^ t)
        Q >>= 1
    X1 = X1 ^ X0
    X2 = X2 ^ X1
    s = X2
    s = s ^ (s >> 1)
    s = s ^ (s >> 2)
    s = s ^ (s >> 4)
    s = s ^ (s >> 8)
    t = s >> 1
    X0, X1, X2 = X0 ^ t, X1 ^ t, X2 ^ t

    def spread(v):
        v = v & jnp.uint32(0x3FF)
        v = (v | (v << 16)) & jnp.uint32(0x030000FF)
        v = (v | (v << 8)) & jnp.uint32(0x0300F00F)
        v = (v | (v << 4)) & jnp.uint32(0x030C30C3)
        v = (v | (v << 2)) & jnp.uint32(0x09249249)
        return v

    code = (spread(X0) << 2) | (spread(X1) << 1) | spread(X2)
    o_ref[...] = lax.bitcast_convert_type(code, jnp.int32)


def _codes_tc(x, interpret=False):
    b, n, _ = x.shape
    x3 = jnp.transpose(x, (2, 0, 1))
    cb = 8
    grid = (b // cb,)
    return pl.pallas_call(
        _codes_body,
        grid=grid,
        in_specs=[pl.BlockSpec((3, cb, n), lambda i: (0, i, 0))],
        out_specs=pl.BlockSpec((cb, n), lambda i: (i, 0)),
        out_shape=jax.ShapeDtypeStruct((b, n), jnp.int32),
        compiler_params=pltpu.CompilerParams(
            dimension_semantics=("parallel",)
        ),
        interpret=interpret,
    )(x3)


def _make_sc_kernel(nc, ns):
    nw = nc * ns
    rpw = NB // nw
    mesh = plsc.VectorSubcoreMesh(core_axis_name="c", subcore_axis_name="s")

    @functools.partial(
        pl.kernel,
        mesh=mesh,
        out_type=[
            jax.ShapeDtypeStruct((NB, N * 3), jnp.float32),
            jax.ShapeDtypeStruct((NB, (N // PATCH) * 3), jnp.float32),
        ],
        compiler_params=pltpu.CompilerParams(needs_layout_passes=False),
        scratch_types=[
            pltpu.VMEM((N,), jnp.int32),
            pltpu.VMEM((N,), jnp.int32),
            pltpu.VMEM((N,), jnp.int32),
            pltpu.VMEM((N,), jnp.int32),
            pltpu.VMEM((BINS * 16,), jnp.int32),
            pltpu.VMEM((BINS * 16,), jnp.int32),
            pltpu.VMEM((N * 3,), jnp.float32),
            pltpu.VMEM((N * 3,), jnp.float32),
            pltpu.VMEM(((N // PATCH) * 3,), jnp.float32),
        ],
    )
    def sc_kernel(codes_hbm, x_hbm, p_hbm, c_hbm,
                  keys_a, keys_b, vals_a, vals_b, hist, base, xrow, prow, crow):
        lane = lax.iota(jnp.int32, 16)
        ones = jnp.zeros((16,), jnp.int32) + 1
        lane0 = lane == 0
        wid = lax.axis_index("s") * nc + lax.axis_index("c")
        zeros16 = jnp.zeros((16,), jnp.int32)

        def zero_body(i, c):
            hist[pl.ds(i * 16, 16)] = zeros16
            return c

        lax.fori_loop(0, BINS, zero_body, 0)

        def do_pass(keys_in, vals_in, keys_out, vals_out, shift, first, last):
            def hist_body(i, c):
                k = keys_in[pl.ds(i * 16, 16)]
                d = ((k >> shift) & (BINS - 1)) * 16 + lane
                plsc.addupdate_scatter(hist, [d], ones)
                return c

            lax.fori_loop(0, NV, hist_body, 0, unroll=4)

            def scan_body(i, carry):
                v = hist[pl.ds(i * 16, 16)]
                cs = plsc.cumsum(v)
                tot = jnp.sum(v)
                hist[pl.ds(i * 16, 16)] = zeros16
                base[pl.ds(i * 16, 16)] = cs - v + carry
                return carry + tot

            lax.fori_loop(0, BINS, scan_body, jnp.int32(0), unroll=4)

            def perm_body(i, c):
                k = keys_in[pl.ds(i * 16, 16)]
                if first:
                    v = i * 16 + lane
                else:
                    v = vals_in[pl.ds(i * 16, 16)]
                d = ((k >> shift) & (BINS - 1)) * 16 + lane
                t = plsc.load_gather(base, [d])
                plsc.store_scatter(base, [d], t + 1)
                if last:
                    plsc.store_scatter(vals_out, [t], v)
                else:
                    a = (t & (NV - 1)) * 16 + (t >> 9)
                    plsc.store_scatter(keys_out, [a], k)
                    plsc.store_scatter(vals_out, [a], v)
                return c

            lax.fori_loop(0, NV, perm_body, 0, unroll=4)

        inv = jnp.float32(1.0 / PATCH)
        lane3 = lane * 3

        def patch_body(j, c):
            s0 = jnp.zeros((16,), jnp.float32)
            s1 = jnp.zeros((16,), jnp.float32)
            s2 = jnp.zeros((16,), jnp.float32)
            for h in range(2):
                idx = vals_a[pl.ds(j * 32 + h * 16, 16)]
                i3 = idx * 3
                t3 = j * 96 + h * 48 + lane3
                vx0 = plsc.load_gather(xrow, [i3])
                vx1 = plsc.load_gather(xrow, [i3 + 1])
                vx2 = plsc.load_gather(xrow, [i3 + 2])
                plsc.store_scatter(prow, [t3], vx0)
                plsc.store_scatter(prow, [t3 + 1], vx1)
                plsc.store_scatter(prow, [t3 + 2], vx2)
                s0 = s0 + vx0
                s1 = s1 + vx1
                s2 = s2 + vx2
            for coord, s in ((0, s0), (1, s1), (2, s2)):
                m = jnp.sum(s) * inv
                plsc.store_scatter(
                    crow,
                    [jnp.zeros((16,), jnp.int32) + (j * 3 + coord)],
                    jnp.zeros((16,), jnp.float32) + m,
                    mask=lane0,
                )
            return c

        def row_body(rr, c):
            row = wid * rpw + rr
            pltpu.sync_copy(codes_hbm.at[row], keys_a)
            pltpu.sync_copy(x_hbm.at[row], xrow)
            do_pass(keys_a, None, keys_b, vals_b, 0, True, False)
            do_pass(keys_b, vals_b, keys_a, vals_a, 8, False, False)
            do_pass(keys_a, vals_a, keys_b, vals_b, 16, False, False)
            do_pass(keys_b, vals_b, keys_a, vals_a, 24, False, True)
            lax.fori_loop(0, N // PATCH, patch_body, 0, unroll=4)
            pltpu.sync_copy(prow, p_hbm.at[row])
            pltpu.sync_copy(crow, c_hbm.at[row])
            return c

        lax.fori_loop(0, rpw, row_body, 0)

    return sc_kernel


def kernel(x):
    b, n, _ = x.shape
    codes = _codes_tc(x)
    info = plsc.get_sparse_core_info()
    sc = _make_sc_kernel(info.num_cores, info.num_subcores)
    xf = x.reshape(b, n * 3)
    p_flat, c_flat = sc(codes, xf)
    s = n // PATCH
    return p_flat.reshape(b, s, PATCH, 3), c_flat.reshape(b, s, 3)


# speedup vs baseline: 1.0756x; 1.0756x over previous
import functools

import jax
import jax.numpy as jnp
from jax import lax
from jax.experimental import pallas as pl
from jax.experimental.pallas import tpu as pltpu
from jax.experimental.pallas import tpu_sc as plsc

BITS = 10
PATCH = 32
N = 8192
NB = 256
BINS = 256
NV = N // 16


def _codes_body(x_ref, o_ref):
    X0f, X1f, X2f = x_ref[0], x_ref[1], x_ref[2]
    maxv = (1 << BITS) - 1
    Xs = []
    for xf in (X0f, X1f, X2f):
        mn = xf.min(axis=-1, keepdims=True)
        mx = xf.max(axis=-1, keepdims=True)
        p = (xf - mn) / jnp.maximum(mx - mn, 1e-9)
        Xs.append(
            jnp.clip((p * (1 << BITS)).astype(jnp.int32), 0, maxv).astype(jnp.uint32)
        )
    X0, X1, X2 = Xs
    Q = 1 << (BITS - 1)
    while Q > 1:
        P = jnp.uint32(Q - 1)
        c0 = (X0 & Q) > 0
        X0 = jnp.where(c0, X0 ^ P, X0)
        c1 = (X1 & Q) > 0
        t = (X0 ^ X1) & P
        X0, X1 = jnp.where(c1, X0 ^ P, X0 ^ t), jnp.where(c1, X1, X1 ^ t)
        c2 = (X2 & Q) > 0
        t = (X0 ^ X2) & P
        X0, X2 = jnp.where(c2, X0 ^ P, X0 ^ t), jnp.where(c2, X2, X2 ^ t)
        Q >>= 1
    X1 = X1 ^ X0
    X2 = X2 ^ X1
    s = X2
    s = s ^ (s >> 1)
    s = s ^ (s >> 2)
    s = s ^ (s >> 4)
    s = s ^ (s >> 8)
    t = s >> 1
    X0, X1, X2 = X0 ^ t, X1 ^ t, X2 ^ t

    def spread(v):
        v = v & jnp.uint32(0x3FF)
        v = (v | (v << 16)) & jnp.uint32(0x030000FF)
        v = (v | (v << 8)) & jnp.uint32(0x0300F00F)
        v = (v | (v << 4)) & jnp.uint32(0x030C30C3)
        v = (v | (v << 2)) & jnp.uint32(0x09249249)
        return v

    code = (spread(X0) << 2) | (spread(X1) << 1) | spread(X2)
    o_ref[...] = lax.bitcast_convert_type(code, jnp.int32)


def _codes_tc(x, interpret=False):
    b, n, _ = x.shape
    x3 = jnp.transpose(x, (2, 0, 1))
    cb = 8
    grid = (b // cb,)
    return pl.pallas_call(
        _codes_body,
        grid=grid,
        in_specs=[pl.BlockSpec((3, cb, n), lambda i: (0, i, 0))],
        out_specs=pl.BlockSpec((cb, n), lambda i: (i, 0)),
        out_shape=jax.ShapeDtypeStruct((b, n), jnp.int32),
        compiler_params=pltpu.CompilerParams(
            dimension_semantics=("parallel",)
        ),
        interpret=interpret,
    )(x3)


def _make_sc_kernel(nc, ns):
    nw = nc * ns
    rpw = NB // nw
    mesh = plsc.VectorSubcoreMesh(core_axis_name="c", subcore_axis_name="s")

    @functools.partial(
        pl.kernel,
        mesh=mesh,
        out_type=[
            jax.ShapeDtypeStruct((NB, N * 3), jnp.float32),
            jax.ShapeDtypeStruct((NB, (N // PATCH) * 3), jnp.float32),
        ],
        compiler_params=pltpu.CompilerParams(needs_layout_passes=False),
        scratch_types=[
            pltpu.VMEM((N,), jnp.int32),
            pltpu.VMEM((N,), jnp.int32),
            pltpu.VMEM((N,), jnp.int32),
            pltpu.VMEM((N,), jnp.int32),
            pltpu.VMEM((BINS * 16,), jnp.int32),
            pltpu.VMEM((BINS * 16,), jnp.int32),
            pltpu.VMEM((N * 3,), jnp.float32),
            pltpu.VMEM((N * 3,), jnp.float32),
            pltpu.VMEM(((N // PATCH) * 3,), jnp.float32),
            pltpu.SemaphoreType.DMA,
        ],
    )
    def sc_kernel(codes_hbm, x_hbm, p_hbm, c_hbm,
                  keys_a, keys_b, vals_a, vals_b, hist, base, xrow, prow, crow,
                  semx):
        lane = lax.iota(jnp.int32, 16)
        ones = jnp.zeros((16,), jnp.int32) + 1
        lane0 = lane == 0
        wid = lax.axis_index("s") * nc + lax.axis_index("c")
        zeros16 = jnp.zeros((16,), jnp.int32)
        nbins = (BINS, BINS, BINS, 64)
        shifts = (0, 8, 16, 24)

        def zero_body(i, c):
            hist[pl.ds(i * 16, 16)] = zeros16
            return c

        lax.fori_loop(0, BINS, zero_body, 0)

        def do_pass(keys_in, vals_in, keys_out, vals_out, pi):
            nb, shift = nbins[pi], shifts[pi]
            first, last = pi == 0, pi == 3

            def hist_body(i, c):
                k = keys_in[pl.ds(i * 16, 16)]
                d = ((k >> shift) & (nb - 1)) * 16 + lane
                plsc.addupdate_scatter(hist, [d], ones)
                return c

            lax.fori_loop(0, NV, hist_body, 0, unroll=4)

            def scan_body(i, carry):
                v = hist[pl.ds(i * 16, 16)]
                cs = plsc.cumsum(v)
                tot = jnp.sum(v)
                hist[pl.ds(i * 16, 16)] = zeros16
                base[pl.ds(i * 16, 16)] = cs - v + carry
                return carry + tot

            lax.fori_loop(0, nb, scan_body, jnp.int32(0))

            def perm_body(i, c):
                k = keys_in[pl.ds(i * 16, 16)]
                if first:
                    v = i * 16 + lane
                else:
                    v = vals_in[pl.ds(i * 16, 16)]
                d = ((k >> shift) & (nb - 1)) * 16 + lane
                t = plsc.load_gather(base, [d])
                plsc.store_scatter(base, [d], t + 1)
                if last:
                    plsc.store_scatter(vals_out, [t], v)
                else:
                    a = (t & (NV - 1)) * 16 + (t >> 9)
                    plsc.store_scatter(keys_out, [a], k)
                    plsc.store_scatter(vals_out, [a], v)
                return c

            lax.fori_loop(0, NV, perm_body, 0, unroll=2)

        inv = jnp.float32(1.0 / PATCH)
        lane3 = lane * 3

        def patch_body(j, c):
            s0 = jnp.zeros((16,), jnp.float32)
            s1 = jnp.zeros((16,), jnp.float32)
            s2 = jnp.zeros((16,), jnp.float32)
            for h in range(2):
                idx = vals_a[pl.ds(j * 32 + h * 16, 16)]
                i3 = idx * 3
                t3 = j * 96 + h * 48 + lane3
                vx0 = plsc.load_gather(xrow, [i3])
                vx1 = plsc.load_gather(xrow, [i3 + 1])
                vx2 = plsc.load_gather(xrow, [i3 + 2])
                plsc.store_scatter(prow, [t3], vx0)
                plsc.store_scatter(prow, [t3 + 1], vx1)
                plsc.store_scatter(prow, [t3 + 2], vx2)
                s0 = s0 + vx0
                s1 = s1 + vx1
                s2 = s2 + vx2
            for coord, s in ((0, s0), (1, s1), (2, s2)):
                m = jnp.sum(s) * inv
                plsc.store_scatter(
                    crow,
                    [jnp.zeros((16,), jnp.int32) + (j * 3 + coord)],
                    jnp.zeros((16,), jnp.float32) + m,
                    mask=lane0,
                )
            return c

        def row_body(rr, c):
            row = wid * rpw + rr
            pltpu.sync_copy(codes_hbm.at[row], keys_a)
            cpx = pltpu.async_copy(x_hbm.at[row], xrow, semx)
            do_pass(keys_a, None, keys_b, vals_b, 0)
            do_pass(keys_b, vals_b, keys_a, vals_a, 1)
            do_pass(keys_a, vals_a, keys_b, vals_b, 2)
            do_pass(keys_b, vals_b, keys_a, vals_a, 3)
            cpx.wait()
            lax.fori_loop(0, N // PATCH, patch_body, 0, unroll=2)
            pltpu.sync_copy(prow, p_hbm.at[row])
            pltpu.sync_copy(crow, c_hbm.at[row])
            return c

        lax.fori_loop(0, rpw, row_body, 0)

    return sc_kernel


def kernel(x):
    b, n, _ = x.shape
    codes = _codes_tc(x)
    info = plsc.get_sparse_core_info()
    sc = _make_sc_kernel(info.num_cores, info.num_subcores)
    xf = x.reshape(b, n * 3)
    p_flat, c_flat = sc(codes, xf)
    s = n // PATCH
    return p_flat.reshape(b, s, PATCH, 3), c_flat.reshape(b, s, 3)


# speedup vs baseline: 1.0831x; 1.0070x over previous
import functools

import jax
import jax.numpy as jnp
from jax import lax
from jax.experimental import pallas as pl
from jax.experimental.pallas import tpu as pltpu
from jax.experimental.pallas import tpu_sc as plsc

BITS = 10
PATCH = 32
N = 8192
NB = 256
BINS = 256
NV = N // 16


def _codes_body(x_ref, o_ref):
    X0f, X1f, X2f = x_ref[0], x_ref[1], x_ref[2]
    maxv = (1 << BITS) - 1
    Xs = []
    for xf in (X0f, X1f, X2f):
        mn = xf.min(axis=-1, keepdims=True)
        mx = xf.max(axis=-1, keepdims=True)
        p = (xf - mn) / jnp.maximum(mx - mn, 1e-9)
        Xs.append(
            jnp.clip((p * (1 << BITS)).astype(jnp.int32), 0, maxv).astype(jnp.uint32)
        )
    X0, X1, X2 = Xs
    Q = 1 << (BITS - 1)
    while Q > 1:
        P = jnp.uint32(Q - 1)
        c0 = (X0 & Q) > 0
        X0 = jnp.where(c0, X0 ^ P, X0)
        c1 = (X1 & Q) > 0
        t = (X0 ^ X1) & P
        X0, X1 = jnp.where(c1, X0 ^ P, X0 ^ t), jnp.where(c1, X1, X1 ^ t)
        c2 = (X2 & Q) > 0
        t = (X0 ^ X2) & P
        X0, X2 = jnp.where(c2, X0 ^ P, X0 ^ t), jnp.where(c2, X2, X2 ^ t)
        Q >>= 1
    X1 = X1 ^ X0
    X2 = X2 ^ X1
    s = X2
    s = s ^ (s >> 1)
    s = s ^ (s >> 2)
    s = s ^ (s >> 4)
    s = s ^ (s >> 8)
    t = s >> 1
    X0, X1, X2 = X0 ^ t, X1 ^ t, X2 ^ t

    def spread(v):
        v = v & jnp.uint32(0x3FF)
        v = (v | (v << 16)) & jnp.uint32(0x030000FF)
        v = (v | (v << 8)) & jnp.uint32(0x0300F00F)
        v = (v | (v << 4)) & jnp.uint32(0x030C30C3)
        v = (v | (v << 2)) & jnp.uint32(0x09249249)
        return v

    code = (spread(X0) << 2) | (spread(X1) << 1) | spread(X2)
    o_ref[...] = lax.bitcast_convert_type(code, jnp.int32)


def _codes_tc(x, interpret=False):
    b, n, _ = x.shape
    x3 = jnp.transpose(x, (2, 0, 1))
    cb = 8
    grid = (b // cb,)
    return pl.pallas_call(
        _codes_body,
        grid=grid,
        in_specs=[pl.BlockSpec((3, cb, n), lambda i: (0, i, 0))],
        out_specs=pl.BlockSpec((cb, n), lambda i: (i, 0)),
        out_shape=jax.ShapeDtypeStruct((b, n), jnp.int32),
        compiler_params=pltpu.CompilerParams(
            dimension_semantics=("parallel",)
        ),
        interpret=interpret,
    )(x3)


def _make_sc_kernel(nc, ns):
    nw = nc * ns
    rpw = NB // nw
    mesh = plsc.VectorSubcoreMesh(core_axis_name="c", subcore_axis_name="s")

    @functools.partial(
        pl.kernel,
        mesh=mesh,
        out_type=[
            jax.ShapeDtypeStruct((NB, N * 3), jnp.float32),
            jax.ShapeDtypeStruct((NB, (N // PATCH) * 3), jnp.float32),
        ],
        compiler_params=pltpu.CompilerParams(needs_layout_passes=False),
        scratch_types=[
            pltpu.VMEM((N,), jnp.int32),
            pltpu.VMEM((N,), jnp.int32),
            pltpu.VMEM((N,), jnp.int32),
            pltpu.VMEM((N,), jnp.int32),
            pltpu.VMEM((BINS * 16,), jnp.int32),
            pltpu.VMEM((BINS * 16,), jnp.int32),
            pltpu.VMEM((N * 3,), jnp.float32),
            pltpu.VMEM((N * 3,), jnp.float32),
            pltpu.VMEM(((N // PATCH) * 3,), jnp.float32),
            pltpu.SemaphoreType.DMA,
            pltpu.SemaphoreType.DMA,
            pltpu.SemaphoreType.DMA,
        ],
    )
    def sc_kernel(codes_hbm, x_hbm, p_hbm, c_hbm,
                  keys_a, keys_b, vals_a, vals_b, hist, base, xrow, prow, crow,
                  semx, semp, semc):
        lane = lax.iota(jnp.int32, 16)
        ones = jnp.zeros((16,), jnp.int32) + 1
        lane0 = lane == 0
        wid = lax.axis_index("s") * nc + lax.axis_index("c")
        zeros16 = jnp.zeros((16,), jnp.int32)
        nbins = (BINS, BINS, BINS, 64)
        shifts = (0, 8, 16, 24)

        def zero_body(i, c):
            hist[pl.ds(i * 16, 16)] = zeros16
            return c

        lax.fori_loop(0, BINS, zero_body, 0)

        def do_pass(keys_in, vals_in, keys_out, vals_out, pi):
            nb, shift = nbins[pi], shifts[pi]
            first, last = pi == 0, pi == 3

            def hist_body(i, c):
                k = keys_in[pl.ds(i * 16, 16)]
                d = ((k >> shift) & (nb - 1)) * 16 + lane
                plsc.addupdate_scatter(hist, [d], ones)
                return c

            lax.fori_loop(0, NV, hist_body, 0, unroll=4)

            def scan_body(i, carry):
                v = hist[pl.ds(i * 16, 16)]
                cs = plsc.cumsum(v)
                tot = jnp.sum(v)
                hist[pl.ds(i * 16, 16)] = zeros16
                base[pl.ds(i * 16, 16)] = cs - v + carry
                return carry + tot

            lax.fori_loop(0, nb, scan_body, jnp.int32(0))

            def perm_body(i, c):
                k = keys_in[pl.ds(i * 16, 16)]
                if first:
                    v = i * 16 + lane
                else:
                    v = vals_in[pl.ds(i * 16, 16)]
                d = ((k >> shift) & (nb - 1)) * 16 + lane
                t = plsc.load_gather(base, [d])
                plsc.store_scatter(base, [d], t + 1)
                if last:
                    plsc.store_scatter(vals_out, [t], v)
                else:
                    a = (t & (NV - 1)) * 16 + (t >> 9)
                    plsc.store_scatter(keys_out, [a], k)
                    plsc.store_scatter(vals_out, [a], v)
                return c

            lax.fori_loop(0, NV, perm_body, 0, unroll=2)

        inv = jnp.float32(1.0 / PATCH)
        lane3 = lane * 3

        def patch_body(j, c):
            s0 = jnp.zeros((16,), jnp.float32)
            s1 = jnp.zeros((16,), jnp.float32)
            s2 = jnp.zeros((16,), jnp.float32)
            for h in range(2):
                idx = vals_a[pl.ds(j * 32 + h * 16, 16)]
                i3 = idx * 3
                t3 = j * 96 + h * 48 + lane3
                vx0 = plsc.load_gather(xrow, [i3])
                vx1 = plsc.load_gather(xrow, [i3 + 1])
                vx2 = plsc.load_gather(xrow, [i3 + 2])
                plsc.store_scatter(prow, [t3], vx0)
                plsc.store_scatter(prow, [t3 + 1], vx1)
                plsc.store_scatter(prow, [t3 + 2], vx2)
                s0 = s0 + vx0
                s1 = s1 + vx1
                s2 = s2 + vx2
            for coord, s in ((0, s0), (1, s1), (2, s2)):
                m = jnp.sum(s) * inv
                plsc.store_scatter(
                    crow,
                    [jnp.zeros((16,), jnp.int32) + (j * 3 + coord)],
                    jnp.zeros((16,), jnp.float32) + m,
                    mask=lane0,
                )
            return c

        def row_body(rr, c):
            row = wid * rpw + rr
            pltpu.sync_copy(codes_hbm.at[row], keys_a)
            cpx = pltpu.async_copy(x_hbm.at[row], xrow, semx)
            do_pass(keys_a, None, keys_b, vals_b, 0)
            do_pass(keys_b, vals_b, keys_a, vals_a, 1)
            do_pass(keys_a, vals_a, keys_b, vals_b, 2)
            do_pass(keys_b, vals_b, keys_a, vals_a, 3)
            cpx.wait()

            @pl.when(rr > 0)
            def _():
                pltpu.make_async_copy(prow, p_hbm.at[row], semp).wait()
                pltpu.make_async_copy(crow, c_hbm.at[row], semc).wait()

            lax.fori_loop(0, N // PATCH, patch_body, 0, unroll=2)
            pltpu.async_copy(prow, p_hbm.at[row], semp)
            pltpu.async_copy(crow, c_hbm.at[row], semc)
            return c

        lax.fori_loop(0, rpw, row_body, 0)
        last_row = wid * rpw + rpw - 1
        pltpu.make_async_copy(prow, p_hbm.at[last_row], semp).wait()
        pltpu.make_async_copy(crow, c_hbm.at[last_row], semc).wait()

    return sc_kernel


def kernel(x):
    b, n, _ = x.shape
    codes = _codes_tc(x)
    info = plsc.get_sparse_core_info()
    sc = _make_sc_kernel(info.num_cores, info.num_subcores)
    xf = x.reshape(b, n * 3)
    p_flat, c_flat = sc(codes, xf)
    s = n // PATCH
    return p_flat.reshape(b, s, PATCH, 3), c_flat.reshape(b, s, 3)


# speedup vs baseline: 1.0840x; 1.0008x over previous
import functools

import jax
import jax.numpy as jnp
from jax import lax
from jax.experimental import pallas as pl
from jax.experimental.pallas import tpu as pltpu
from jax.experimental.pallas import tpu_sc as plsc

BITS = 10
PATCH = 32
N = 8192
NB = 256
BINS = 256
NV = N // 16


def _codes_body(x_ref, o_ref):
    X0f, X1f, X2f = x_ref[0], x_ref[1], x_ref[2]
    maxv = (1 << BITS) - 1
    Xs = []
    for xf in (X0f, X1f, X2f):
        mn = xf.min(axis=-1, keepdims=True)
        mx = xf.max(axis=-1, keepdims=True)
        p = (xf - mn) / jnp.maximum(mx - mn, 1e-9)
        Xs.append(
            jnp.clip((p * (1 << BITS)).astype(jnp.int32), 0, maxv).astype(jnp.uint32)
        )
    X0, X1, X2 = Xs
    Q = 1 << (BITS - 1)
    while Q > 1:
        P = jnp.uint32(Q - 1)
        c0 = (X0 & Q) > 0
        X0 = jnp.where(c0, X0 ^ P, X0)
        c1 = (X1 & Q) > 0
        t = (X0 ^ X1) & P
        X0, X1 = jnp.where(c1, X0 ^ P, X0 ^ t), jnp.where(c1, X1, X1 ^ t)
        c2 = (X2 & Q) > 0
        t = (X0 ^ X2) & P
        X0, X2 = jnp.where(c2, X0 ^ P, X0 ^ t), jnp.where(c2, X2, X2 ^ t)
        Q >>= 1
    X1 = X1 ^ X0
    X2 = X2 ^ X1
    s = X2
    s = s ^ (s >> 1)
    s = s ^ (s >> 2)
    s = s ^ (s >> 4)
    s = s ^ (s >> 8)
    t = s >> 1
    X0, X1, X2 = X0 ^ t, X1 ^ t, X2 ^ t

    def spread(v):
        v = v & jnp.uint32(0x3FF)
        v = (v | (v << 16)) & jnp.uint32(0x030000FF)
        v = (v | (v << 8)) & jnp.uint32(0x0300F00F)
        v = (v | (v << 4)) & jnp.uint32(0x030C30C3)
        v = (v | (v << 2)) & jnp.uint32(0x09249249)
        return v

    code = (spread(X0) << 2) | (spread(X1) << 1) | spread(X2)
    o_ref[...] = lax.bitcast_convert_type(code, jnp.int32)


def _codes_tc(x, interpret=False):
    b, n, _ = x.shape
    x3 = jnp.transpose(x, (2, 0, 1))
    cb = 8
    grid = (b // cb,)
    return pl.pallas_call(
        _codes_body,
        grid=grid,
        in_specs=[pl.BlockSpec((3, cb, n), lambda i: (0, i, 0))],
        out_specs=pl.BlockSpec((cb, n), lambda i: (i, 0)),
        out_shape=jax.ShapeDtypeStruct((b, n), jnp.int32),
        compiler_params=pltpu.CompilerParams(
            dimension_semantics=("parallel",)
        ),
        interpret=interpret,
    )(x3)


def _make_sc_kernel(nc, ns):
    nw = nc * ns
    rpw = NB // nw
    mesh = plsc.VectorSubcoreMesh(core_axis_name="c", subcore_axis_name="s")

    @functools.partial(
        pl.kernel,
        mesh=mesh,
        out_type=[
            jax.ShapeDtypeStruct((NB, N * 3), jnp.float32),
            jax.ShapeDtypeStruct((NB, (N // PATCH) * 3), jnp.float32),
        ],
        compiler_params=pltpu.CompilerParams(
            needs_layout_passes=False, use_tc_tiling_on_sc=True
        ),
        scratch_types=[
            pltpu.VMEM((N,), jnp.int32),
            pltpu.VMEM((N,), jnp.int32),
            pltpu.VMEM((N,), jnp.int32),
            pltpu.VMEM((N,), jnp.int32),
            pltpu.VMEM((BINS * 16,), jnp.int32),
            pltpu.VMEM((BINS * 16,), jnp.int32),
            pltpu.VMEM((N * 3,), jnp.float32),
            pltpu.VMEM((N * 3,), jnp.float32),
            pltpu.VMEM(((N // PATCH) * 3,), jnp.float32),
            pltpu.SemaphoreType.DMA,
            pltpu.SemaphoreType.DMA,
            pltpu.SemaphoreType.DMA,
        ],
    )
    def sc_kernel(codes_hbm, x_hbm, p_hbm, c_hbm,
                  keys_a, keys_b, vals_a, vals_b, hist, base, xrow, prow, crow,
                  semx, semp, semc):
        lane = lax.iota(jnp.int32, 16)
        ones = jnp.zeros((16,), jnp.int32) + 1
        lane0 = lane == 0
        wid = lax.axis_index("s") * nc + lax.axis_index("c")
        zeros16 = jnp.zeros((16,), jnp.int32)
        nbins = (BINS, BINS, BINS, 64)
        shifts = (0, 8, 16, 24)

        def zero_body(i, c):
            hist[pl.ds(i * 16, 16)] = zeros16
            return c

        lax.fori_loop(0, BINS, zero_body, 0)

        def do_pass(keys_in, vals_in, keys_out, vals_out, pi):
            nb, shift = nbins[pi], shifts[pi]
            first, last = pi == 0, pi == 3

            def hist_body(i, c):
                k = keys_in[pl.ds(i * 16, 16)]
                d = ((k >> shift) & (nb - 1)) * 16 + lane
                plsc.addupdate_scatter(hist, [d], ones)
                return c

            lax.fori_loop(0, NV, hist_body, 0, unroll=4)

            def scan_body(i, carry):
                v = hist[pl.ds(i * 16, 16)]
                cs = plsc.cumsum(v)
                tot = jnp.sum(v)
                hist[pl.ds(i * 16, 16)] = zeros16
                base[pl.ds(i * 16, 16)] = cs - v + carry
                return carry + tot

            lax.fori_loop(0, nb, scan_body, jnp.int32(0))

            def perm_body(i, c):
                k = keys_in[pl.ds(i * 16, 16)]
                if first:
                    v = i * 16 + lane
                else:
                    v = vals_in[pl.ds(i * 16, 16)]
                d = ((k >> shift) & (nb - 1)) * 16 + lane
                t = plsc.load_gather(base, [d])
                plsc.store_scatter(base, [d], t + 1)
                if last:
                    plsc.store_scatter(vals_out, [t], v)
                else:
                    a = (t & (NV - 1)) * 16 + (t >> 9)
                    plsc.store_scatter(keys_out, [a], k)
                    plsc.store_scatter(vals_out, [a], v)
                return c

            lax.fori_loop(0, NV, perm_body, 0, unroll=2)

        inv = jnp.float32(1.0 / PATCH)
        lane3 = lane * 3

        def patch_body(j, c):
            s0 = jnp.zeros((16,), jnp.float32)
            s1 = jnp.zeros((16,), jnp.float32)
            s2 = jnp.zeros((16,), jnp.float32)
            for h in range(2):
                idx = vals_a[pl.ds(j * 32 + h * 16, 16)]
                i3 = idx * 3
                t3 = j * 96 + h * 48 + lane3
                vx0 = plsc.load_gather(xrow, [i3])
                vx1 = plsc.load_gather(xrow, [i3 + 1])
                vx2 = plsc.load_gather(xrow, [i3 + 2])
                plsc.store_scatter(prow, [t3], vx0)
                plsc.store_scatter(prow, [t3 + 1], vx1)
                plsc.store_scatter(prow, [t3 + 2], vx2)
                s0 = s0 + vx0
                s1 = s1 + vx1
                s2 = s2 + vx2
            for coord, s in ((0, s0), (1, s1), (2, s2)):
                m = jnp.sum(s) * inv
                plsc.store_scatter(
                    crow,
                    [jnp.zeros((16,), jnp.int32) + (j * 3 + coord)],
                    jnp.zeros((16,), jnp.float32) + m,
                    mask=lane0,
                )
            return c

        def row_body(rr, c):
            row = wid * rpw + rr
            pltpu.sync_copy(codes_hbm.at[row], keys_a)
            cpx = pltpu.async_copy(x_hbm.at[row], xrow, semx)
            do_pass(keys_a, None, keys_b, vals_b, 0)
            do_pass(keys_b, vals_b, keys_a, vals_a, 1)
            do_pass(keys_a, vals_a, keys_b, vals_b, 2)
            do_pass(keys_b, vals_b, keys_a, vals_a, 3)
            cpx.wait()

            @pl.when(rr > 0)
            def _():
                pltpu.make_async_copy(prow, p_hbm.at[row], semp).wait()
                pltpu.make_async_copy(crow, c_hbm.at[row], semc).wait()

            lax.fori_loop(0, N // PATCH, patch_body, 0, unroll=2)
            pltpu.async_copy(prow, p_hbm.at[row], semp)
            pltpu.async_copy(crow, c_hbm.at[row], semc)
            return c

        lax.fori_loop(0, rpw, row_body, 0)
        last_row = wid * rpw + rpw - 1
        pltpu.make_async_copy(prow, p_hbm.at[last_row], semp).wait()
        pltpu.make_async_copy(crow, c_hbm.at[last_row], semc).wait()

    return sc_kernel


def kernel(x):
    b, n, _ = x.shape
    codes = _codes_tc(x)
    info = plsc.get_sparse_core_info()
    sc = _make_sc_kernel(info.num_cores, info.num_subcores)
    xf = x.reshape(b, n * 3)
    p_flat, c_flat = sc(codes, xf)
    s = n // PATCH
    return p_flat.reshape(b, s, PATCH, 3), c_flat.reshape(b, s, 3)


# speedup vs baseline: 1.0974x; 1.0123x over previous
import functools

import jax
import jax.numpy as jnp
from jax import lax
from jax.experimental import pallas as pl
from jax.experimental.pallas import tpu as pltpu
from jax.experimental.pallas import tpu_sc as plsc

BITS = 10
PATCH = 32
N = 8192
NB = 256
BINS = 256
NV = N // 16


def _codes_body(x_ref, o_ref):
    X0f, X1f, X2f = x_ref[0], x_ref[1], x_ref[2]
    maxv = (1 << BITS) - 1
    Xs = []
    for xf in (X0f, X1f, X2f):
        mn = xf.min(axis=-1, keepdims=True)
        mx = xf.max(axis=-1, keepdims=True)
        p = (xf - mn) / jnp.maximum(mx - mn, 1e-9)
        Xs.append(
            jnp.clip((p * (1 << BITS)).astype(jnp.int32), 0, maxv).astype(jnp.uint32)
        )
    X0, X1, X2 = Xs
    Q = 1 << (BITS - 1)
    while Q > 1:
        P = jnp.uint32(Q - 1)
        c0 = (X0 & Q) > 0
        X0 = jnp.where(c0, X0 ^ P, X0)
        c1 = (X1 & Q) > 0
        t = (X0 ^ X1) & P
        X0, X1 = jnp.where(c1, X0 ^ P, X0 ^ t), jnp.where(c1, X1, X1 ^ t)
        c2 = (X2 & Q) > 0
        t = (X0 ^ X2) & P
        X0, X2 = jnp.where(c2, X0 ^ P, X0 ^ t), jnp.where(c2, X2, X2 ^ t)
        Q >>= 1
    X1 = X1 ^ X0
    X2 = X2 ^ X1
    s = X2
    s = s ^ (s >> 1)
    s = s ^ (s >> 2)
    s = s ^ (s >> 4)
    s = s ^ (s >> 8)
    t = s >> 1
    X0, X1, X2 = X0 ^ t, X1 ^ t, X2 ^ t

    def spread(v):
        v = v & jnp.uint32(0x3FF)
        v = (v | (v << 16)) & jnp.uint32(0x030000FF)
        v = (v | (v << 8)) & jnp.uint32(0x0300F00F)
        v = (v | (v << 4)) & jnp.uint32(0x030C30C3)
        v = (v | (v << 2)) & jnp.uint32(0x09249249)
        return v

    code = (spread(X0) << 2) | (spread(X1) << 1) | spread(X2)
    o_ref[...] = lax.bitcast_convert_type(code, jnp.int32)


def _codes_tc(x, interpret=False):
    b, n, _ = x.shape
    x3 = jnp.transpose(x, (2, 0, 1))
    cb = 8
    grid = (b // cb,)
    return pl.pallas_call(
        _codes_body,
        grid=grid,
        in_specs=[pl.BlockSpec((3, cb, n), lambda i: (0, i, 0))],
        out_specs=pl.BlockSpec((cb, n), lambda i: (i, 0)),
        out_shape=jax.ShapeDtypeStruct((b, n), jnp.int32),
        compiler_params=pltpu.CompilerParams(
            dimension_semantics=("parallel",)
        ),
        interpret=interpret,
    )(x3)


def _make_sc_kernel(nc, ns):
    nw = nc * ns
    rpw = NB // nw
    mesh = plsc.VectorSubcoreMesh(core_axis_name="c", subcore_axis_name="s")

    @functools.partial(
        pl.kernel,
        mesh=mesh,
        out_type=[
            jax.ShapeDtypeStruct((NB, N * 3), jnp.float32),
            jax.ShapeDtypeStruct((NB, (N // PATCH) * 3), jnp.float32),
        ],
        compiler_params=pltpu.CompilerParams(
            needs_layout_passes=False, use_tc_tiling_on_sc=True
        ),
        scratch_types=[
            pltpu.VMEM((N,), jnp.int32),
            pltpu.VMEM((N,), jnp.int32),
            pltpu.VMEM((N,), jnp.int32),
            pltpu.VMEM((N,), jnp.int32),
            pltpu.VMEM((BINS * 16,), jnp.int32),
            pltpu.VMEM((BINS * 16,), jnp.int32),
            pltpu.VMEM((N * 3,), jnp.float32),
            pltpu.VMEM((N * 3,), jnp.float32),
            pltpu.VMEM(((N // PATCH) * 3,), jnp.float32),
            pltpu.VMEM((N,), jnp.int32),
            pltpu.VMEM((N,), jnp.int32),
            pltpu.SemaphoreType.DMA,
            pltpu.SemaphoreType.DMA,
            pltpu.SemaphoreType.DMA,
            pltpu.SemaphoreType.DMA,
        ],
    )
    def sc_kernel(codes_hbm, x_hbm, p_hbm, c_hbm,
                  keys_a, keys_b, vals_a, vals_b, hist, base, xrow, prow, crow,
                  cbuf0, cbuf1, semx, semp, semc, semk):
        lane = lax.iota(jnp.int32, 16)
        ones = jnp.zeros((16,), jnp.int32) + 1
        lane0 = lane == 0
        wid = lax.axis_index("s") * nc + lax.axis_index("c")
        zeros16 = jnp.zeros((16,), jnp.int32)
        nbins = (BINS, BINS, BINS, 64)
        shifts = (0, 8, 16, 24)

        def zero_body(i, c):
            hist[pl.ds(i * 16, 16)] = zeros16
            return c

        lax.fori_loop(0, BINS, zero_body, 0)

        def do_pass(keys_in, vals_in, keys_out, vals_out, pi):
            nb, shift = nbins[pi], shifts[pi]
            first, last = pi == 0, pi == 3

            def hist_body(i, c):
                k = keys_in[pl.ds(i * 16, 16)]
                d = ((k >> shift) & (nb - 1)) * 16 + lane
                plsc.addupdate_scatter(hist, [d], ones)
                return c

            lax.fori_loop(0, NV, hist_body, 0, unroll=4)

            def scan_body(i, carry):
                v = hist[pl.ds(i * 16, 16)]
                cs = plsc.cumsum(v)
                tot = jnp.sum(v)
                hist[pl.ds(i * 16, 16)] = zeros16
                base[pl.ds(i * 16, 16)] = cs - v + carry
                return carry + tot

            lax.fori_loop(0, nb, scan_body, jnp.int32(0))

            def perm_body(i, c):
                k = keys_in[pl.ds(i * 16, 16)]
                if first:
                    v = i * 16 + lane
                else:
                    v = vals_in[pl.ds(i * 16, 16)]
                d = ((k >> shift) & (nb - 1)) * 16 + lane
                t = plsc.load_gather(base, [d])
                plsc.store_scatter(base, [d], t + 1)
                if last:
                    plsc.store_scatter(vals_out, [t], v)
                else:
                    a = (t & (NV - 1)) * 16 + (t >> 9)
                    plsc.store_scatter(keys_out, [a], k)
                    plsc.store_scatter(vals_out, [a], v)
                return c

            lax.fori_loop(0, NV, perm_body, 0, unroll=2)

        inv = jnp.float32(1.0 / PATCH)
        lane3 = lane * 3

        def patch_body(j, c):
            s0 = jnp.zeros((16,), jnp.float32)
            s1 = jnp.zeros((16,), jnp.float32)
            s2 = jnp.zeros((16,), jnp.float32)
            for h in range(2):
                idx = vals_a[pl.ds(j * 32 + h * 16, 16)]
                i3 = idx * 3
                t3 = j * 96 + h * 48 + lane3
                vx0 = plsc.load_gather(xrow, [i3])
                vx1 = plsc.load_gather(xrow, [i3 + 1])
                vx2 = plsc.load_gather(xrow, [i3 + 2])
                plsc.store_scatter(prow, [t3], vx0)
                plsc.store_scatter(prow, [t3 + 1], vx1)
                plsc.store_scatter(prow, [t3 + 2], vx2)
                s0 = s0 + vx0
                s1 = s1 + vx1
                s2 = s2 + vx2
            for coord, s in ((0, s0), (1, s1), (2, s2)):
                m = jnp.sum(s) * inv
                plsc.store_scatter(
                    crow,
                    [jnp.zeros((16,), jnp.int32) + (j * 3 + coord)],
                    jnp.zeros((16,), jnp.float32) + m,
                    mask=lane0,
                )
            return c

        def half_row(row, cbuf, first_row):
            cpx = pltpu.async_copy(x_hbm.at[row], xrow, semx)
            do_pass(cbuf, None, keys_b, vals_b, 0)
            do_pass(keys_b, vals_b, keys_a, vals_a, 1)
            do_pass(keys_a, vals_a, keys_b, vals_b, 2)
            do_pass(keys_b, vals_b, keys_a, vals_a, 3)
            cpx.wait()

            if first_row:
                @pl.when(row > wid * rpw)
                def _():
                    pltpu.make_async_copy(prow, p_hbm.at[row], semp).wait()
                    pltpu.make_async_copy(crow, c_hbm.at[row], semc).wait()
            else:
                pltpu.make_async_copy(prow, p_hbm.at[row], semp).wait()
                pltpu.make_async_copy(crow, c_hbm.at[row], semc).wait()

            lax.fori_loop(0, N // PATCH, patch_body, 0, unroll=2)
            pltpu.async_copy(prow, p_hbm.at[row], semp)
            pltpu.async_copy(crow, c_hbm.at[row], semc)

        npair = rpw // 2
        pltpu.sync_copy(codes_hbm.at[wid * rpw], cbuf0)

        def pair_body(q, c):
            r0 = wid * rpw + 2 * q

            @pl.when(q > 0)
            def _():
                pltpu.make_async_copy(codes_hbm.at[r0], cbuf0, semk).wait()

            pltpu.async_copy(codes_hbm.at[r0 + 1], cbuf1, semk)
            half_row(r0, cbuf0, True)
            pltpu.make_async_copy(codes_hbm.at[r0 + 1], cbuf1, semk).wait()

            @pl.when(q < npair - 1)
            def _():
                pltpu.async_copy(codes_hbm.at[r0 + 2], cbuf0, semk)

            half_row(r0 + 1, cbuf1, False)
            return c

        lax.fori_loop(0, npair, pair_body, 0)
        last_row = wid * rpw + rpw - 1
        pltpu.make_async_copy(prow, p_hbm.at[last_row], semp).wait()
        pltpu.make_async_copy(crow, c_hbm.at[last_row], semc).wait()

    return sc_kernel


def kernel(x):
    b, n, _ = x.shape
    codes = _codes_tc(x)
    info = plsc.get_sparse_core_info()
    sc = _make_sc_kernel(info.num_cores, info.num_subcores)
    xf = x.reshape(b, n * 3)
    p_flat, c_flat = sc(codes, xf)
    s = n // PATCH
    return p_flat.reshape(b, s, PATCH, 3), c_flat.reshape(b, s, 3)
